# Initial kernel scaffold; baseline (speedup 1.0000x reference)
import jax
import jax.numpy as jnp
from jax.experimental import pallas as pl


def kernel(overlaps, scores, attributes):
    raise NotImplementedError("write your pallas kernel here")


# speedup vs baseline: 4.1379x; 4.1379x over previous
import functools

import jax
import jax.numpy as jnp
from jax import lax
from jax.experimental import pallas as pl
from jax.experimental.pallas import tpu as pltpu
from jax.experimental.pallas import tpu_sc as plsc

MAX_OUT = 200
OVERLAP_T = 0.9
SCORE_T = 0.1
N = 5000
NPAD = 5024
APAD = NPAD * 4
C = 320
NCH = C // 16
OUT_PAD = 256
GATH_PAD = OUT_PAD * 4
NEG = float("-inf")
BIG = 1 << 30


def _nms_body(ov_hbm, sc_hbm, attr_hbm, gath_out, mask_out, sel_out,
              s_v, row_v, sel_v, mask_v, gsel_v, rb_v, pair_v, rows_v,
              attr_v, sh, done_s):
    b = lax.axis_index("c")
    sid = lax.axis_index("s")
    a_t = jnp.minimum(sid * C, N - C)
    lane = lax.iota(jnp.int32, 16)
    neg_v = jnp.full((16,), NEG, jnp.float32)

    pltpu.sync_copy(sc_hbm.at[pl.ds(b * N + a_t, C)], s_v)
    for k in range(NCH):
        v = s_v[pl.ds(16 * k, 16)]
        s_v[pl.ds(16 * k, 16)] = jnp.where(v > SCORE_T, v, neg_v)

    zf = jnp.zeros((16,), jnp.float32)
    zi = jnp.zeros((16,), jnp.int32)
    pad_i = jnp.full((16,), NPAD - 1, jnp.int32)
    for k in range(OUT_PAD // 16):
        sel_v[pl.ds(16 * k, 16)] = zi
        mask_v[pl.ds(16 * k, 16)] = zf
        gsel_v[pl.ds(16 * k, 16)] = pad_i
    done_s[0] = jnp.int32(0)

    def body(i, carry):
        @pl.when(done_s[0] == 0)
        def _it():
            cur = s_v[pl.ds(0, 16)]
            curk = jnp.zeros((16,), jnp.int32)
            for k in range(1, NCH):
                v = s_v[pl.ds(16 * k, 16)]
                better = v > cur
                cur = jnp.where(better, v, cur)
                curk = jnp.where(better, jnp.full((16,), k, jnp.int32), curk)
            cand = a_t + curk * 16 + lane
            mloc = jnp.max(cur)
            loc_best = jnp.min(jnp.where(cur == mloc, cand, BIG))

            pair_v[...] = jnp.where(
                lane == 0, jnp.full((16,), mloc, jnp.float32),
                plsc.bitcast(jnp.full((16,), loc_best, jnp.int32),
                             jnp.float32))
            pltpu.sync_copy(pair_v, sh.at[32 + sid])
            plsc.subcore_barrier()
            pltpu.sync_copy(sh.at[pl.ds(32, 16)], rb_v)
            plsc.subcore_barrier()
            maxes = plsc.load_gather(rb_v, [lane, zi])
            idxs = plsc.bitcast(
                plsc.load_gather(rb_v, [lane, jnp.full((16,), 1, jnp.int32)]),
                jnp.int32)
            mx = jnp.max(maxes)
            best = jnp.min(jnp.where(maxes == mx, idxs, BIG))
            any_active = mx > NEG

            @pl.when(jnp.logical_not(any_active))
            def _stop():
                done_s[0] = jnp.int32(1)

            @pl.when(any_active)
            def _select():
                iv = jnp.full((16,), i, jnp.int32)
                l0 = lane == 0
                plsc.store_scatter(sel_v, [iv],
                                   jnp.full((16,), best, jnp.int32), mask=l0)
                plsc.store_scatter(mask_v, [iv],
                                   jnp.full((16,), 1.0, jnp.float32), mask=l0)
                plsc.store_scatter(gsel_v, [iv],
                                   jnp.full((16,), best, jnp.int32), mask=l0)
                pltpu.sync_copy(
                    ov_hbm.at[pl.ds((b * N + best) * N + a_t, C)], row_v)
                for k in range(NCH):
                    g = a_t + 16 * k + lane
                    ov = row_v[pl.ds(16 * k, 16)]
                    sv = s_v[pl.ds(16 * k, 16)]
                    kill = jnp.logical_or(ov > OVERLAP_T, g == best)
                    s_v[pl.ds(16 * k, 16)] = jnp.where(kill, neg_v, sv)

        return carry

    lax.fori_loop(0, MAX_OUT, body, jnp.int32(0))

    @pl.when(sid == 0)
    def _out():
        pltpu.sync_copy(sel_v, sel_out.at[pl.ds(b * OUT_PAD, OUT_PAD)])
        pltpu.sync_copy(mask_v, mask_out.at[pl.ds(b * OUT_PAD, OUT_PAD)])
        pltpu.sync_copy(attr_hbm.at[pl.ds(b * APAD, APAD)], attr_v)
        row_of_lane = lane >> 2
        col_of_lane = lane & 3
        for c in range(GATH_PAD // 16):
            r = plsc.load_gather(gsel_v, [4 * c + row_of_lane])
            vals = plsc.load_gather(attr_v, [r * 4 + col_of_lane])
            rows_v[pl.ds(16 * c, 16)] = vals
        pltpu.sync_copy(rows_v, gath_out.at[pl.ds(b * GATH_PAD, GATH_PAD)])


@jax.jit
def _nms_call(overlaps_flat, scores_flat, attr_flat):
    mesh = plsc.VectorSubcoreMesh(core_axis_name="c", subcore_axis_name="s",
                                  num_cores=2, num_subcores=16)
    f = functools.partial(
        pl.kernel,
        out_type=(
            jax.ShapeDtypeStruct((2 * GATH_PAD,), jnp.float32),
            jax.ShapeDtypeStruct((2 * OUT_PAD,), jnp.float32),
            jax.ShapeDtypeStruct((2 * OUT_PAD,), jnp.int32),
        ),
        mesh=mesh,
        scratch_types=[
            pltpu.VMEM((C,), jnp.float32),
            pltpu.VMEM((C,), jnp.float32),
            pltpu.VMEM((OUT_PAD,), jnp.int32),
            pltpu.VMEM((OUT_PAD,), jnp.float32),
            pltpu.VMEM((OUT_PAD,), jnp.int32),
            pltpu.VMEM((16, 16), jnp.float32),
            pltpu.VMEM((16,), jnp.float32),
            pltpu.VMEM((GATH_PAD,), jnp.float32),
            pltpu.VMEM((APAD,), jnp.float32),
            pltpu.VMEM_SHARED((64, 16), jnp.float32),
            pltpu.SMEM((1,), jnp.int32),
        ],
        compiler_params=pltpu.CompilerParams(needs_layout_passes=False),
    )(_nms_body)
    return f(overlaps_flat, scores_flat, attr_flat)


def kernel(overlaps, scores, attributes):
    attr_pad = jnp.concatenate(
        [attributes,
         jnp.zeros((attributes.shape[0], NPAD - N, attributes.shape[2]),
                   attributes.dtype)], axis=1)
    gath, mask, sel = _nms_call(
        overlaps.reshape(-1), scores.reshape(-1), attr_pad.reshape(-1))
    gathered = gath.reshape(2, OUT_PAD, 4)[:, :MAX_OUT]
    mask = mask.reshape(2, OUT_PAD)[:, :MAX_OUT]
    sel = sel.reshape(2, OUT_PAD)[:, :MAX_OUT]
    return (gathered, mask, sel)


# speedup vs baseline: 9.0178x; 2.1793x over previous
import functools

import jax
import jax.numpy as jnp
from jax import lax
from jax.experimental import pallas as pl
from jax.experimental.pallas import tpu as pltpu
from jax.experimental.pallas import tpu_sc as plsc

MAX_OUT = 200
OVERLAP_T = 0.9
SCORE_T = 0.1
N = 5000
NPAD = 5024
APAD = NPAD * 4
W = 384
WSTEP = 384
CMAX = 4608
NCH = 25
OUT_PAD = 256
GATH_PAD = OUT_PAD * 4
NEG = float("-inf")
BIG = 1 << 30


def _nms_body(ov_hbm, sc_hbm, attr_hbm, gath_out, mask_out, sel_out,
              s_v, rbuf_v, sel_v, mask_v, gsel_v, rb_v, pair_v, rows_v,
              attr_v, tail2_v, sh, done_s):
    b = lax.axis_index("c")
    sid = lax.axis_index("s")
    c0 = pl.multiple_of(jnp.minimum(WSTEP * sid, CMAX), 128)
    lane = lax.iota(jnp.int32, 16)
    neg_v = jnp.full((16,), NEG, jnp.float32)

    pltpu.sync_copy(sc_hbm.at[pl.ds(b * N + c0, W)], s_v.at[pl.ds(0, W)])
    for k in range(NCH - 1):
        v = s_v[pl.ds(16 * k, 16)]
        s_v[pl.ds(16 * k, 16)] = jnp.where(v > SCORE_T, v, neg_v)
    s_v[pl.ds(16 * (NCH - 1), 16)] = neg_v

    @pl.when(sid == 15)
    def _tail_scores():
        pltpu.sync_copy(sc_hbm.at[pl.ds(b * N + 4992, 8)],
                        s_v.at[pl.ds(16 * (NCH - 1), 8)])
        vt = s_v[pl.ds(16 * (NCH - 1), 16)]
        s_v[pl.ds(16 * (NCH - 1), 16)] = jnp.where(
            jnp.logical_and(vt > SCORE_T, lane < 8), vt, neg_v)

    zf = jnp.zeros((16,), jnp.float32)
    zi = jnp.zeros((16,), jnp.int32)
    pad_i = jnp.full((16,), NPAD - 1, jnp.int32)
    for k in range(OUT_PAD // 16):
        sel_v[pl.ds(16 * k, 16)] = zi
        mask_v[pl.ds(16 * k, 16)] = zf
        gsel_v[pl.ds(16 * k, 16)] = pad_i
    done_s[0] = jnp.int32(0)

    def body(i, carry):
        @pl.when(done_s[0] == 0)
        def _it():
            cur = s_v[pl.ds(0, 16)]
            curk = jnp.zeros((16,), jnp.int32)
            for k in range(1, NCH):
                v = s_v[pl.ds(16 * k, 16)]
                better = v > cur
                cur = jnp.where(better, v, cur)
                curk = jnp.where(better, jnp.full((16,), k, jnp.int32), curk)
            cand = c0 + curk * 16 + lane
            mloc = jnp.max(cur)
            loc_best = jnp.min(jnp.where(cur == mloc, cand, BIG))

            pair_v[...] = jnp.where(
                lane == 0, jnp.full((16,), mloc, jnp.float32),
                plsc.bitcast(jnp.full((16,), loc_best, jnp.int32),
                             jnp.float32))
            pltpu.sync_copy(pair_v, sh.at[32 + sid])
            plsc.subcore_barrier()
            pltpu.sync_copy(sh.at[pl.ds(32, 16)], rb_v)
            plsc.subcore_barrier()
            maxes = plsc.load_gather(rb_v, [lane, zi])
            idxs = plsc.bitcast(
                plsc.load_gather(rb_v, [lane, jnp.full((16,), 1, jnp.int32)]),
                jnp.int32)
            mx = jnp.max(maxes)
            best = jnp.min(jnp.where(maxes == mx, idxs, BIG))
            any_active = mx > NEG

            @pl.when(jnp.logical_not(any_active))
            def _stop():
                done_s[0] = jnp.int32(1)

            @pl.when(any_active)
            def _select():
                iv = jnp.full((16,), i, jnp.int32)
                l0 = lane == 0
                plsc.store_scatter(sel_v, [iv],
                                   jnp.full((16,), best, jnp.int32), mask=l0)
                plsc.store_scatter(mask_v, [iv],
                                   jnp.full((16,), 1.0, jnp.float32), mask=l0)
                plsc.store_scatter(gsel_v, [iv],
                                   jnp.full((16,), best, jnp.int32), mask=l0)
                rp = pl.multiple_of((best >> 1) * 2, 2)
                odd = (best & 1) == 1
                pltpu.sync_copy(
                    ov_hbm.at[b, pl.ds(rp, 2), pl.ds(c0, W)], rbuf_v)
                for k in range(NCH - 1):
                    g = c0 + 16 * k + lane
                    ov0 = rbuf_v[0, pl.ds(16 * k, 16)]
                    ov1 = rbuf_v[1, pl.ds(16 * k, 16)]
                    ov = jnp.where(odd, ov1, ov0)
                    sv = s_v[pl.ds(16 * k, 16)]
                    kill = jnp.logical_or(ov > OVERLAP_T, g == best)
                    s_v[pl.ds(16 * k, 16)] = jnp.where(kill, neg_v, sv)

                @pl.when(sid == 15)
                def _tail_suppress():
                    pltpu.sync_copy(
                        ov_hbm.at[b, pl.ds(rp, 2), pl.ds(4992, 8)], tail2_v)
                    rowi = jnp.full((16,), jnp.where(odd, 1, 0), jnp.int32)
                    coli = jnp.where(lane < 8, lane, 0)
                    ovt = plsc.load_gather(tail2_v, [rowi, coli])
                    g = 4992 + lane
                    sv = s_v[pl.ds(16 * (NCH - 1), 16)]
                    kill = jnp.logical_or(
                        jnp.logical_and(ovt > OVERLAP_T, lane < 8),
                        g == best)
                    s_v[pl.ds(16 * (NCH - 1), 16)] = jnp.where(kill, neg_v, sv)

        return carry

    lax.fori_loop(0, MAX_OUT, body, jnp.int32(0))

    @pl.when(sid == 0)
    def _out():
        pltpu.sync_copy(sel_v, sel_out.at[pl.ds(b * OUT_PAD, OUT_PAD)])
        pltpu.sync_copy(mask_v, mask_out.at[pl.ds(b * OUT_PAD, OUT_PAD)])
        pltpu.sync_copy(attr_hbm.at[pl.ds(b * APAD, APAD)], attr_v)
        row_of_lane = lane >> 2
        col_of_lane = lane & 3
        for c in range(GATH_PAD // 16):
            r = plsc.load_gather(gsel_v, [4 * c + row_of_lane])
            vals = plsc.load_gather(attr_v, [r * 4 + col_of_lane])
            rows_v[pl.ds(16 * c, 16)] = vals
        pltpu.sync_copy(rows_v, gath_out.at[pl.ds(b * GATH_PAD, GATH_PAD)])


@jax.jit
def _nms_call(overlaps, scores_flat, attr_flat):
    mesh = plsc.VectorSubcoreMesh(core_axis_name="c", subcore_axis_name="s",
                                  num_cores=2, num_subcores=16)
    f = functools.partial(
        pl.kernel,
        out_type=(
            jax.ShapeDtypeStruct((2 * GATH_PAD,), jnp.float32),
            jax.ShapeDtypeStruct((2 * OUT_PAD,), jnp.float32),
            jax.ShapeDtypeStruct((2 * OUT_PAD,), jnp.int32),
        ),
        mesh=mesh,
        scratch_types=[
            pltpu.VMEM((16 * NCH,), jnp.float32),
            pltpu.VMEM((2, W), jnp.float32),
            pltpu.VMEM((OUT_PAD,), jnp.int32),
            pltpu.VMEM((OUT_PAD,), jnp.float32),
            pltpu.VMEM((OUT_PAD,), jnp.int32),
            pltpu.VMEM((16, 16), jnp.float32),
            pltpu.VMEM((16,), jnp.float32),
            pltpu.VMEM((GATH_PAD,), jnp.float32),
            pltpu.VMEM((APAD,), jnp.float32),
            pltpu.VMEM((2, 8), jnp.float32),
            pltpu.VMEM_SHARED((64, 16), jnp.float32),
            pltpu.SMEM((1,), jnp.int32),
        ],
        compiler_params=pltpu.CompilerParams(needs_layout_passes=False),
    )(_nms_body)
    return f(overlaps, scores_flat, attr_flat)


def kernel(overlaps, scores, attributes):
    attr_pad = jnp.concatenate(
        [attributes,
         jnp.zeros((attributes.shape[0], NPAD - N, attributes.shape[2]),
                   attributes.dtype)], axis=1)
    gath, mask, sel = _nms_call(
        overlaps, scores.reshape(-1), attr_pad.reshape(-1))
    gathered = gath.reshape(2, OUT_PAD, 4)[:, :MAX_OUT]
    mask = mask.reshape(2, OUT_PAD)[:, :MAX_OUT]
    sel = sel.reshape(2, OUT_PAD)[:, :MAX_OUT]
    return (gathered, mask, sel)


# speedup vs baseline: 15.2318x; 1.6891x over previous
import functools

import jax
import jax.numpy as jnp
from jax import lax
from jax.experimental import pallas as pl
from jax.experimental.pallas import tpu as pltpu
from jax.experimental.pallas import tpu_sc as plsc

MAX_OUT = 200
OVERLAP_T = 0.9
SCORE_T = 0.1
N = 5000
NPAD = 5024
APAD = NPAD * 4
W = 384
WSTEP = 384
CMAX = 4608
NCH = 25
OUT_PAD = 256
GATH_PAD = OUT_PAD * 4
NEG = float("-inf")
BIG = 1 << 30

DONE, SRC, ODD, PREV, PENDA, PENDB, IDXA, IDXB = range(8)


def _nms_body(ov_hbm, sc_hbm, attr_hbm, gath_out, mask_out, sel_out,
              s_v, rf_v, ra_v, rb2_v, sel_v, mask_v, gsel_v, rb_v, pair_v,
              rows_v, attr_v, tf_v, ta_v, tb_v, sh, st_s,
              semf, sema, semb):
    b = lax.axis_index("c")
    sid = lax.axis_index("s")
    c0 = pl.multiple_of(jnp.minimum(WSTEP * sid, CMAX), 128)
    lane = lax.iota(jnp.int32, 16)
    neg_v = jnp.full((16,), NEG, jnp.float32)

    pltpu.sync_copy(sc_hbm.at[pl.ds(b * N + c0, W)], s_v.at[pl.ds(0, W)])
    for k in range(NCH - 1):
        v = s_v[pl.ds(16 * k, 16)]
        s_v[pl.ds(16 * k, 16)] = jnp.where(v > SCORE_T, v, neg_v)
    s_v[pl.ds(16 * (NCH - 1), 16)] = neg_v

    @pl.when(sid == 15)
    def _tail_scores():
        pltpu.sync_copy(sc_hbm.at[pl.ds(b * N + 4992, 8)],
                        s_v.at[pl.ds(16 * (NCH - 1), 8)])
        vt = s_v[pl.ds(16 * (NCH - 1), 16)]
        s_v[pl.ds(16 * (NCH - 1), 16)] = jnp.where(
            jnp.logical_and(vt > SCORE_T, lane < 8), vt, neg_v)

    zf = jnp.zeros((16,), jnp.float32)
    zi = jnp.zeros((16,), jnp.int32)
    pad_i = jnp.full((16,), NPAD - 1, jnp.int32)
    for k in range(OUT_PAD // 16):
        sel_v[pl.ds(16 * k, 16)] = zi
        mask_v[pl.ds(16 * k, 16)] = zf
        gsel_v[pl.ds(16 * k, 16)] = pad_i
    for j in range(8):
        st_s[j] = jnp.int32(0)

    bufs = {1: (rf_v, tf_v, semf), 2: (ra_v, ta_v, sema), 3: (rb2_v, tb_v, semb)}

    def _drain(code):
        rbuf, tbuf, sem = bufs[code]
        pltpu.make_async_copy(
            ov_hbm.at[b, pl.ds(0, 2), pl.ds(c0, W)], rbuf, sem).wait()

        @pl.when(sid == 15)
        def _():
            pltpu.make_async_copy(
                ov_hbm.at[b, pl.ds(0, 2), pl.ds(4992, 8)], tbuf, sem).wait()

    def _issue(code, ridx):
        rbuf, tbuf, sem = bufs[code]
        rp = pl.multiple_of((ridx >> 1) * 2, 2)
        pltpu.async_copy(ov_hbm.at[b, pl.ds(rp, 2), pl.ds(c0, W)], rbuf, sem)

        @pl.when(sid == 15)
        def _():
            pltpu.async_copy(
                ov_hbm.at[b, pl.ds(rp, 2), pl.ds(4992, 8)], tbuf, sem)

    def body(i, carry):
        @pl.when(st_s[DONE] == 0)
        def _it():
            src = st_s[SRC]
            odd = st_s[ODD] == 1
            prev = st_s[PREV]

            def fused(code):
                cur = None
                curk = jnp.zeros((16,), jnp.int32)
                for k in range(NCH):
                    sv = s_v[pl.ds(16 * k, 16)]
                    if code:
                        rbuf, tbuf, _ = bufs[code]
                        if k < NCH - 1:
                            ov0 = rbuf[0, pl.ds(16 * k, 16)]
                            ov1 = rbuf[1, pl.ds(16 * k, 16)]
                            ovv = jnp.where(odd, ov1, ov0)
                            g = c0 + 16 * k + lane
                            kill = jnp.logical_or(ovv > OVERLAP_T, g == prev)
                            sv = jnp.where(kill, neg_v, sv)
                            s_v[pl.ds(16 * k, 16)] = sv
                        else:
                            rowi = jnp.full((16,), jnp.where(odd, 1, 0),
                                            jnp.int32)
                            coli = jnp.where(lane < 8, lane, 0)
                            ovt = plsc.load_gather(tbuf, [rowi, coli])
                            g = 4992 + lane
                            kill = jnp.logical_or(
                                jnp.logical_and(ovt > OVERLAP_T, lane < 8),
                                g == prev)
                            sv = jnp.where(kill, neg_v, sv)
                            s_v[pl.ds(16 * k, 16)] = sv
                    if cur is None:
                        cur = sv
                    else:
                        better = sv > cur
                        cur = jnp.where(better, sv, cur)
                        curk = jnp.where(better,
                                         jnp.full((16,), k, jnp.int32), curk)
                cand = c0 + curk * 16 + lane
                mloc = jnp.max(cur)
                loc_best = jnp.min(jnp.where(cur == mloc, cand, BIG))
                pair_v[...] = jnp.where(
                    lane == 0, jnp.full((16,), mloc, jnp.float32),
                    plsc.bitcast(jnp.full((16,), loc_best, jnp.int32),
                                 jnp.float32))

            for code in (0, 1, 2, 3):
                @pl.when(src == code)
                def _var(code=code):
                    if code:
                        _drain(code)
                    fused(code)

            par = i & 1
            bank = pl.multiple_of(32 + 16 * par, 16)
            pltpu.sync_copy(pair_v, sh.at[bank + sid])
            plsc.subcore_barrier()
            pltpu.sync_copy(sh.at[pl.ds(bank, 16)], rb_v)
            maxes = plsc.load_gather(rb_v, [lane, zi])
            idxs = plsc.bitcast(
                plsc.load_gather(rb_v, [lane, jnp.full((16,), 1, jnp.int32)]),
                jnp.int32)
            mx = jnp.max(maxes)
            best = jnp.min(jnp.where(maxes == mx, idxs, BIG))
            any_active = mx > NEG

            @pl.when(jnp.logical_not(any_active))
            def _stop():
                st_s[DONE] = jnp.int32(1)
                st_s[SRC] = jnp.int32(0)

            @pl.when(any_active)
            def _select():
                lastA = par == 1
                lastpend = jnp.where(lastA, st_s[PENDA], st_s[PENDB])
                lastidx = jnp.where(lastA, st_s[IDXA], st_s[IDXB])
                hit = jnp.logical_and(lastpend == 1, lastidx == best)

                @pl.when(hit)
                def _hit():
                    st_s[SRC] = jnp.where(lastA, 2, 3)
                    st_s[PENDA] = jnp.where(lastA, 0, st_s[PENDA])
                    st_s[PENDB] = jnp.where(lastA, st_s[PENDB], 0)

                @pl.when(jnp.logical_not(hit))
                def _miss():
                    _issue(1, best)
                    st_s[SRC] = jnp.int32(1)

                st_s[ODD] = best & 1
                st_s[PREV] = best
                iv = jnp.full((16,), i, jnp.int32)
                l0 = lane == 0
                plsc.store_scatter(sel_v, [iv],
                                   jnp.full((16,), best, jnp.int32), mask=l0)
                plsc.store_scatter(mask_v, [iv],
                                   jnp.full((16,), 1.0, jnp.float32), mask=l0)
                plsc.store_scatter(gsel_v, [iv],
                                   jnp.full((16,), best, jnp.int32), mask=l0)

                m2v = jnp.where(idxs == best, neg_v, maxes)
                mx2 = jnp.max(m2v)
                spec = jnp.min(jnp.where(m2v == mx2, idxs, BIG))

                @pl.when(mx2 > NEG)
                def _spec():
                    curA = par == 0

                    @pl.when(jnp.logical_and(curA, st_s[PENDA] == 1))
                    def _dra():
                        _drain(2)

                    @pl.when(jnp.logical_and(jnp.logical_not(curA),
                                             st_s[PENDB] == 1))
                    def _drb():
                        _drain(3)

                    @pl.when(curA)
                    def _ia():
                        _issue(2, spec)
                        st_s[PENDA] = jnp.int32(1)
                        st_s[IDXA] = spec

                    @pl.when(jnp.logical_not(curA))
                    def _ib():
                        _issue(3, spec)
                        st_s[PENDB] = jnp.int32(1)
                        st_s[IDXB] = spec

        return carry

    lax.fori_loop(0, MAX_OUT, body, jnp.int32(0))

    for code, slot in ((1, SRC), (2, PENDA), (3, PENDB)):
        @pl.when((st_s[slot] == code) if code == 1 else (st_s[slot] == 1))
        def _fd(code=code):
            _drain(code)

    @pl.when(jnp.logical_or(st_s[SRC] == 2, st_s[SRC] == 3))
    def _fd_src_spec():
        @pl.when(st_s[SRC] == 2)
        def _():
            _drain(2)

        @pl.when(st_s[SRC] == 3)
        def _():
            _drain(3)

    @pl.when(sid == 0)
    def _out():
        pltpu.sync_copy(sel_v, sel_out.at[pl.ds(b * OUT_PAD, OUT_PAD)])
        pltpu.sync_copy(mask_v, mask_out.at[pl.ds(b * OUT_PAD, OUT_PAD)])
        pltpu.sync_copy(attr_hbm.at[pl.ds(b * APAD, APAD)], attr_v)
        row_of_lane = lane >> 2
        col_of_lane = lane & 3
        for c in range(GATH_PAD // 16):
            r = plsc.load_gather(gsel_v, [4 * c + row_of_lane])
            vals = plsc.load_gather(attr_v, [r * 4 + col_of_lane])
            rows_v[pl.ds(16 * c, 16)] = vals
        pltpu.sync_copy(rows_v, gath_out.at[pl.ds(b * GATH_PAD, GATH_PAD)])


@jax.jit
def _nms_call(overlaps, scores_flat, attr_flat):
    mesh = plsc.VectorSubcoreMesh(core_axis_name="c", subcore_axis_name="s",
                                  num_cores=2, num_subcores=16)
    f = functools.partial(
        pl.kernel,
        out_type=(
            jax.ShapeDtypeStruct((2 * GATH_PAD,), jnp.float32),
            jax.ShapeDtypeStruct((2 * OUT_PAD,), jnp.float32),
            jax.ShapeDtypeStruct((2 * OUT_PAD,), jnp.int32),
        ),
        mesh=mesh,
        scratch_types=[
            pltpu.VMEM((16 * NCH,), jnp.float32),
            pltpu.VMEM((2, W), jnp.float32),
            pltpu.VMEM((2, W), jnp.float32),
            pltpu.VMEM((2, W), jnp.float32),
            pltpu.VMEM((OUT_PAD,), jnp.int32),
            pltpu.VMEM((OUT_PAD,), jnp.float32),
            pltpu.VMEM((OUT_PAD,), jnp.int32),
            pltpu.VMEM((16, 16), jnp.float32),
            pltpu.VMEM((16,), jnp.float32),
            pltpu.VMEM((GATH_PAD,), jnp.float32),
            pltpu.VMEM((APAD,), jnp.float32),
            pltpu.VMEM((2, 8), jnp.float32),
            pltpu.VMEM((2, 8), jnp.float32),
            pltpu.VMEM((2, 8), jnp.float32),
            pltpu.VMEM_SHARED((64, 16), jnp.float32),
            pltpu.SMEM((8,), jnp.int32),
            pltpu.SemaphoreType.DMA,
            pltpu.SemaphoreType.DMA,
            pltpu.SemaphoreType.DMA,
        ],
        compiler_params=pltpu.CompilerParams(needs_layout_passes=False),
    )(_nms_body)
    return f(overlaps, scores_flat, attr_flat)


def kernel(overlaps, scores, attributes):
    attr_pad = jnp.concatenate(
        [attributes,
         jnp.zeros((attributes.shape[0], NPAD - N, attributes.shape[2]),
                   attributes.dtype)], axis=1)
    gath, mask, sel = _nms_call(
        overlaps, scores.reshape(-1), attr_pad.reshape(-1))
    gathered = gath.reshape(2, OUT_PAD, 4)[:, :MAX_OUT]
    mask = mask.reshape(2, OUT_PAD)[:, :MAX_OUT]
    sel = sel.reshape(2, OUT_PAD)[:, :MAX_OUT]
    return (gathered, mask, sel)


# speedup vs baseline: 15.5881x; 1.0234x over previous
import functools

import jax
import jax.numpy as jnp
from jax import lax
from jax.experimental import pallas as pl
from jax.experimental.pallas import tpu as pltpu
from jax.experimental.pallas import tpu_sc as plsc

MAX_OUT = 200
OVERLAP_T = 0.9
SCORE_T = 0.1
N = 5000
NPAD = 5024
APAD = NPAD * 4
W = 384
WSTEP = 384
CMAX = 4608
NCH = 25
OUT_PAD = 256
GATH_PAD = OUT_PAD * 4
NEG = float("-inf")
BIG = 1 << 30

DONE, SRC, ODD, PREV, PENDA, PENDB, IDXA, IDXB = range(8)


def _nms_body(ov_hbm, sc_hbm, attr_hbm, gath_out, mask_out, sel_out,
              s_v, rf_v, ra_v, rb2_v, sel_v, mask_v, gsel_v, rb_v, pair_v,
              rows_v, attr_v, tf_v, ta_v, tb_v, sh, st_s,
              semf, sema, semb):
    b = lax.axis_index("c")
    sid = lax.axis_index("s")
    c0 = pl.multiple_of(jnp.minimum(WSTEP * sid, CMAX), 128)
    lane = lax.iota(jnp.int32, 16)
    neg_v = jnp.full((16,), NEG, jnp.float32)

    pltpu.sync_copy(sc_hbm.at[pl.ds(b * N + c0, W)], s_v.at[pl.ds(0, W)])
    for k in range(NCH - 1):
        v = s_v[pl.ds(16 * k, 16)]
        s_v[pl.ds(16 * k, 16)] = jnp.where(v > SCORE_T, v, neg_v)
    s_v[pl.ds(16 * (NCH - 1), 16)] = neg_v

    @pl.when(sid == 15)
    def _tail_scores():
        pltpu.sync_copy(sc_hbm.at[pl.ds(b * N + 4992, 8)],
                        s_v.at[pl.ds(16 * (NCH - 1), 8)])
        vt = s_v[pl.ds(16 * (NCH - 1), 16)]
        s_v[pl.ds(16 * (NCH - 1), 16)] = jnp.where(
            jnp.logical_and(vt > SCORE_T, lane < 8), vt, neg_v)

    zf = jnp.zeros((16,), jnp.float32)
    zi = jnp.zeros((16,), jnp.int32)
    pad_i = jnp.full((16,), NPAD - 1, jnp.int32)
    for k in range(OUT_PAD // 16):
        sel_v[pl.ds(16 * k, 16)] = zi
        mask_v[pl.ds(16 * k, 16)] = zf
        gsel_v[pl.ds(16 * k, 16)] = pad_i
    for j in range(8):
        st_s[j] = jnp.int32(0)

    bufs = {1: (rf_v, tf_v, semf), 2: (ra_v, ta_v, sema), 3: (rb2_v, tb_v, semb)}

    def _drain(code):
        rbuf, tbuf, sem = bufs[code]
        pltpu.make_async_copy(
            ov_hbm.at[b, pl.ds(0, 2), pl.ds(c0, W)], rbuf, sem).wait()

        @pl.when(sid == 15)
        def _():
            pltpu.make_async_copy(
                ov_hbm.at[b, pl.ds(0, 2), pl.ds(4992, 8)], tbuf, sem).wait()

    def _issue(code, ridx):
        rbuf, tbuf, sem = bufs[code]
        rp = pl.multiple_of((ridx >> 1) * 2, 2)
        pltpu.async_copy(ov_hbm.at[b, pl.ds(rp, 2), pl.ds(c0, W)], rbuf, sem)

        @pl.when(sid == 15)
        def _():
            pltpu.async_copy(
                ov_hbm.at[b, pl.ds(rp, 2), pl.ds(4992, 8)], tbuf, sem)

    def body(i):
        if True:
            src = st_s[SRC]
            oddi = st_s[ODD]
            odd = oddi == 1
            prev = st_s[PREV]

            def fused(code):
                cur = None
                curk = jnp.zeros((16,), jnp.int32)
                for k in range(NCH):
                    sv = s_v[pl.ds(16 * k, 16)]
                    if code:
                        rbuf, tbuf, _ = bufs[code]
                        if k < NCH - 1:
                            ovv = rbuf[oddi, pl.ds(16 * k, 16)]
                            g = c0 + 16 * k + lane
                            kill = jnp.logical_or(ovv > OVERLAP_T, g == prev)
                            sv = jnp.where(kill, neg_v, sv)
                            s_v[pl.ds(16 * k, 16)] = sv
                        else:
                            rowi = jnp.full((16,), jnp.where(odd, 1, 0),
                                            jnp.int32)
                            coli = jnp.where(lane < 8, lane, 0)
                            ovt = plsc.load_gather(tbuf, [rowi, coli])
                            g = 4992 + lane
                            kill = jnp.logical_or(
                                jnp.logical_and(ovt > OVERLAP_T, lane < 8),
                                g == prev)
                            sv = jnp.where(kill, neg_v, sv)
                            s_v[pl.ds(16 * k, 16)] = sv
                    if cur is None:
                        cur = sv
                    else:
                        better = sv > cur
                        cur = jnp.where(better, sv, cur)
                        curk = jnp.where(better,
                                         jnp.full((16,), k, jnp.int32), curk)
                cand = c0 + curk * 16 + lane
                mloc = jnp.max(cur)
                loc_best = jnp.min(jnp.where(cur == mloc, cand, BIG))
                pair_v[...] = jnp.where(
                    lane == 0, jnp.full((16,), mloc, jnp.float32),
                    plsc.bitcast(jnp.full((16,), loc_best, jnp.int32),
                                 jnp.float32))

            for code in (0, 1, 2, 3):
                @pl.when(src == code)
                def _var(code=code):
                    if code:
                        _drain(code)
                    fused(code)

            par = i & 1
            bank = pl.multiple_of(32 + 16 * par, 16)
            pltpu.sync_copy(pair_v, sh.at[bank + sid])
            plsc.subcore_barrier()
            pltpu.sync_copy(sh.at[pl.ds(bank, 16)], rb_v)
            maxes = plsc.load_gather(rb_v, [lane, zi])
            idxs = plsc.bitcast(
                plsc.load_gather(rb_v, [lane, jnp.full((16,), 1, jnp.int32)]),
                jnp.int32)
            mx = jnp.max(maxes)
            best = jnp.min(jnp.where(maxes == mx, idxs, BIG))
            any_active = mx > NEG

            @pl.when(jnp.logical_not(any_active))
            def _stop():
                st_s[DONE] = jnp.int32(1)
                st_s[SRC] = jnp.int32(0)

            @pl.when(any_active)
            def _select():
                lastA = par == 1
                lastpend = jnp.where(lastA, st_s[PENDA], st_s[PENDB])
                lastidx = jnp.where(lastA, st_s[IDXA], st_s[IDXB])
                hit = jnp.logical_and(lastpend == 1, lastidx == best)

                @pl.when(hit)
                def _hit():
                    st_s[SRC] = jnp.where(lastA, 2, 3)
                    st_s[PENDA] = jnp.where(lastA, 0, st_s[PENDA])
                    st_s[PENDB] = jnp.where(lastA, st_s[PENDB], 0)

                @pl.when(jnp.logical_not(hit))
                def _miss():
                    _issue(1, best)
                    st_s[SRC] = jnp.int32(1)

                st_s[ODD] = best & 1
                st_s[PREV] = best
                iv = jnp.full((16,), i, jnp.int32)
                l0 = lane == 0
                plsc.store_scatter(sel_v, [iv],
                                   jnp.full((16,), best, jnp.int32), mask=l0)
                plsc.store_scatter(mask_v, [iv],
                                   jnp.full((16,), 1.0, jnp.float32), mask=l0)
                plsc.store_scatter(gsel_v, [iv],
                                   jnp.full((16,), best, jnp.int32), mask=l0)

                m2v = jnp.where(idxs == best, neg_v, maxes)
                mx2 = jnp.max(m2v)
                spec = jnp.min(jnp.where(m2v == mx2, idxs, BIG))

                @pl.when(mx2 > NEG)
                def _spec():
                    curA = par == 0

                    @pl.when(jnp.logical_and(curA, st_s[PENDA] == 1))
                    def _dra():
                        _drain(2)

                    @pl.when(jnp.logical_and(jnp.logical_not(curA),
                                             st_s[PENDB] == 1))
                    def _drb():
                        _drain(3)

                    @pl.when(curA)
                    def _ia():
                        _issue(2, spec)
                        st_s[PENDA] = jnp.int32(1)
                        st_s[IDXA] = spec

                    @pl.when(jnp.logical_not(curA))
                    def _ib():
                        _issue(3, spec)
                        st_s[PENDB] = jnp.int32(1)
                        st_s[IDXB] = spec

    def _cond(c):
        return jnp.logical_and(c[0] < MAX_OUT, c[1] == 0)

    def _wbody(c):
        body(c[0])
        return (c[0] + 1, st_s[DONE])

    lax.while_loop(_cond, _wbody, (jnp.int32(0), jnp.int32(0)))

    for code, slot in ((1, SRC), (2, PENDA), (3, PENDB)):
        @pl.when((st_s[slot] == code) if code == 1 else (st_s[slot] == 1))
        def _fd(code=code):
            _drain(code)

    @pl.when(jnp.logical_or(st_s[SRC] == 2, st_s[SRC] == 3))
    def _fd_src_spec():
        @pl.when(st_s[SRC] == 2)
        def _():
            _drain(2)

        @pl.when(st_s[SRC] == 3)
        def _():
            _drain(3)

    @pl.when(sid == 0)
    def _out():
        pltpu.sync_copy(sel_v, sel_out.at[pl.ds(b * OUT_PAD, OUT_PAD)])
        pltpu.sync_copy(mask_v, mask_out.at[pl.ds(b * OUT_PAD, OUT_PAD)])
        pltpu.sync_copy(attr_hbm.at[pl.ds(b * APAD, APAD)], attr_v)
        row_of_lane = lane >> 2
        col_of_lane = lane & 3
        for c in range(GATH_PAD // 16):
            r = plsc.load_gather(gsel_v, [4 * c + row_of_lane])
            vals = plsc.load_gather(attr_v, [r * 4 + col_of_lane])
            rows_v[pl.ds(16 * c, 16)] = vals
        pltpu.sync_copy(rows_v, gath_out.at[pl.ds(b * GATH_PAD, GATH_PAD)])


@jax.jit
def _nms_call(overlaps, scores_flat, attr_flat):
    mesh = plsc.VectorSubcoreMesh(core_axis_name="c", subcore_axis_name="s",
                                  num_cores=2, num_subcores=16)
    f = functools.partial(
        pl.kernel,
        out_type=(
            jax.ShapeDtypeStruct((2 * GATH_PAD,), jnp.float32),
            jax.ShapeDtypeStruct((2 * OUT_PAD,), jnp.float32),
            jax.ShapeDtypeStruct((2 * OUT_PAD,), jnp.int32),
        ),
        mesh=mesh,
        scratch_types=[
            pltpu.VMEM((16 * NCH,), jnp.float32),
            pltpu.VMEM((2, W), jnp.float32),
            pltpu.VMEM((2, W), jnp.float32),
            pltpu.VMEM((2, W), jnp.float32),
            pltpu.VMEM((OUT_PAD,), jnp.int32),
            pltpu.VMEM((OUT_PAD,), jnp.float32),
            pltpu.VMEM((OUT_PAD,), jnp.int32),
            pltpu.VMEM((16, 16), jnp.float32),
            pltpu.VMEM((16,), jnp.float32),
            pltpu.VMEM((GATH_PAD,), jnp.float32),
            pltpu.VMEM((APAD,), jnp.float32),
            pltpu.VMEM((2, 8), jnp.float32),
            pltpu.VMEM((2, 8), jnp.float32),
            pltpu.VMEM((2, 8), jnp.float32),
            pltpu.VMEM_SHARED((64, 16), jnp.float32),
            pltpu.SMEM((8,), jnp.int32),
            pltpu.SemaphoreType.DMA,
            pltpu.SemaphoreType.DMA,
            pltpu.SemaphoreType.DMA,
        ],
        compiler_params=pltpu.CompilerParams(needs_layout_passes=False),
    )(_nms_body)
    return f(overlaps, scores_flat, attr_flat)


def kernel(overlaps, scores, attributes):
    attr_pad = jnp.concatenate(
        [attributes,
         jnp.zeros((attributes.shape[0], NPAD - N, attributes.shape[2]),
                   attributes.dtype)], axis=1)
    gath, mask, sel = _nms_call(
        overlaps, scores.reshape(-1), attr_pad.reshape(-1))
    gathered = gath.reshape(2, OUT_PAD, 4)[:, :MAX_OUT]
    mask = mask.reshape(2, OUT_PAD)[:, :MAX_OUT]
    sel = sel.reshape(2, OUT_PAD)[:, :MAX_OUT]
    return (gathered, mask, sel)
